# baseline (speedup 1.0000x reference)
import jax
import jax.numpy as jnp
from jax import lax
from jax.experimental import pallas as pl
from jax.experimental.pallas import tpu as pltpu

_N = 8192
_D = 64
_BLOCK = 128
_NEG = float("-inf")
_G = 8
_C = 128


def _normalize_kernel(x_ref, zn_ref):
    x = x_ref[...]
    norms = jnp.sqrt(jnp.sum(x * x, axis=1, keepdims=True))
    zn_ref[...] = x / jnp.maximum(norms, 1e-12)


def _knn_block_kernel(zn_hbm, out_ref, zn_vmem, sem):
    i = pl.program_id(0)

    @pl.when(i == 0)
    def _():
        cp = pltpu.make_async_copy(zn_hbm, zn_vmem, sem)
        cp.start()
        cp.wait()

    zb = zn_vmem[pl.ds(i * _BLOCK, _BLOCK), :]
    s = lax.dot_general(zb, zn_vmem[...], (((1,), (1,)), ((), ())),
                        preferred_element_type=jnp.float32)
    col = lax.broadcasted_iota(jnp.int32, (_BLOCK, _N), 1)
    row = lax.broadcasted_iota(jnp.int32, (_BLOCK, _N), 0) + i * _BLOCK
    sm = jnp.where(col == row, _NEG, s)

    for g in range(_BLOCK // _G):
        smg = sm[g * _G:(g + 1) * _G, :]
        a = smg[:, 0:_C]
        b = jnp.full((_G, _C), _NEG, jnp.float32)
        for k in range(1, _N // _C):
            x = smg[:, k * _C:(k + 1) * _C]
            t = jnp.minimum(a, x)
            a = jnp.maximum(a, x)
            b = jnp.maximum(b, t)
        v1 = jnp.max(a, axis=1, keepdims=True)
        eq = a == v1
        l2 = jnp.max(jnp.where(eq, _NEG, a), axis=1, keepdims=True)
        bat = jnp.max(jnp.where(eq, b, _NEG), axis=1, keepdims=True)
        v2 = jnp.maximum(l2, bat)
        out_ref[g * _G:(g + 1) * _G, :] = jnp.where(smg >= v2, smg, 0.0)


def kernel(z_x):
    zn = pl.pallas_call(
        _normalize_kernel,
        out_shape=jax.ShapeDtypeStruct((_N, _D), jnp.float32),
    )(z_x)
    return pl.pallas_call(
        _knn_block_kernel,
        grid=(_N // _BLOCK,),
        in_specs=[pl.BlockSpec(memory_space=pltpu.MemorySpace.HBM)],
        out_specs=pl.BlockSpec((_BLOCK, _N), lambda i: (i, 0)),
        out_shape=jax.ShapeDtypeStruct((_N, _N), jnp.float32),
        scratch_shapes=[pltpu.VMEM((_N, _D), jnp.float32),
                        pltpu.SemaphoreType.DMA],
    )(zn)


# speedup vs baseline: 1.1911x; 1.1911x over previous
import jax
import jax.numpy as jnp
from jax import lax
from jax.experimental import pallas as pl
from jax.experimental.pallas import tpu as pltpu

_N = 8192
_D = 64
_BLOCK = 512
_NEG = float("-inf")
_G = 8
_C = 128


def _normalize_kernel(x_ref, zn_ref):
    x = x_ref[...]
    norms = jnp.sqrt(jnp.sum(x * x, axis=1, keepdims=True))
    zn_ref[...] = x / jnp.maximum(norms, 1e-12)


def _knn_block_kernel(zn_hbm, out_ref, zn_vmem, sem):
    i = pl.program_id(0)

    @pl.when(i == 0)
    def _():
        cp = pltpu.make_async_copy(zn_hbm, zn_vmem, sem)
        cp.start()
        cp.wait()

    zb = zn_vmem[pl.ds(i * _BLOCK, _BLOCK), :]
    s = lax.dot_general(zb, zn_vmem[...], (((1,), (1,)), ((), ())),
                        preferred_element_type=jnp.float32)
    col = lax.broadcasted_iota(jnp.int32, (_BLOCK, _N), 1)
    row = lax.broadcasted_iota(jnp.int32, (_BLOCK, _N), 0) + i * _BLOCK
    sm = jnp.where(col == row, _NEG, s)

    for g in range(_BLOCK // _G):
        smg = sm[g * _G:(g + 1) * _G, :]
        a = smg[:, 0:_C]
        b = jnp.full((_G, _C), _NEG, jnp.float32)
        for k in range(1, _N // _C):
            x = smg[:, k * _C:(k + 1) * _C]
            t = jnp.minimum(a, x)
            a = jnp.maximum(a, x)
            b = jnp.maximum(b, t)
        v1 = jnp.max(a, axis=1, keepdims=True)
        eq = a == v1
        l2 = jnp.max(jnp.where(eq, _NEG, a), axis=1, keepdims=True)
        bat = jnp.max(jnp.where(eq, b, _NEG), axis=1, keepdims=True)
        v2 = jnp.maximum(l2, bat)
        out_ref[g * _G:(g + 1) * _G, :] = jnp.where(smg >= v2, smg, 0.0)


def kernel(z_x):
    zn = pl.pallas_call(
        _normalize_kernel,
        out_shape=jax.ShapeDtypeStruct((_N, _D), jnp.float32),
    )(z_x)
    return pl.pallas_call(
        _knn_block_kernel,
        grid=(_N // _BLOCK,),
        in_specs=[pl.BlockSpec(memory_space=pltpu.MemorySpace.HBM)],
        out_specs=pl.BlockSpec((_BLOCK, _N), lambda i: (i, 0)),
        out_shape=jax.ShapeDtypeStruct((_N, _N), jnp.float32),
        scratch_shapes=[pltpu.VMEM((_N, _D), jnp.float32),
                        pltpu.SemaphoreType.DMA],
    )(zn)


# speedup vs baseline: 1.2421x; 1.0428x over previous
import jax
import jax.numpy as jnp
from jax import lax
from jax.experimental import pallas as pl
from jax.experimental.pallas import tpu as pltpu

_N = 8192
_D = 64
_BLOCK = 512
_NEG = float("-inf")
_G = 8
_C = 128


def _knn_block_kernel(x_hbm, out_ref, zn_vmem, sem):
    i = pl.program_id(0)

    @pl.when(i == 0)
    def _():
        cp = pltpu.make_async_copy(x_hbm, zn_vmem, sem)
        cp.start()
        cp.wait()
        x = zn_vmem[...]
        norms = jnp.sqrt(jnp.sum(x * x, axis=1, keepdims=True))
        zn_vmem[...] = x / jnp.maximum(norms, 1e-12)

    zb = zn_vmem[pl.ds(i * _BLOCK, _BLOCK), :]
    s = lax.dot_general(zb, zn_vmem[...], (((1,), (1,)), ((), ())),
                        preferred_element_type=jnp.float32)
    col = lax.broadcasted_iota(jnp.int32, (_BLOCK, _N), 1)
    row = lax.broadcasted_iota(jnp.int32, (_BLOCK, _N), 0) + i * _BLOCK
    sm = jnp.where(col == row, _NEG, s)

    for g in range(_BLOCK // _G):
        smg = sm[g * _G:(g + 1) * _G, :]
        a = smg[:, 0:_C]
        b = jnp.full((_G, _C), _NEG, jnp.float32)
        for k in range(1, _N // _C):
            x = smg[:, k * _C:(k + 1) * _C]
            t = jnp.minimum(a, x)
            a = jnp.maximum(a, x)
            b = jnp.maximum(b, t)
        v1 = jnp.max(a, axis=1, keepdims=True)
        eq = a == v1
        l2 = jnp.max(jnp.where(eq, _NEG, a), axis=1, keepdims=True)
        bat = jnp.max(jnp.where(eq, b, _NEG), axis=1, keepdims=True)
        v2 = jnp.maximum(l2, bat)
        out_ref[g * _G:(g + 1) * _G, :] = jnp.where(smg >= v2, smg, 0.0)


def kernel(z_x):
    return pl.pallas_call(
        _knn_block_kernel,
        grid=(_N // _BLOCK,),
        in_specs=[pl.BlockSpec(memory_space=pltpu.MemorySpace.HBM)],
        out_specs=pl.BlockSpec((_BLOCK, _N), lambda i: (i, 0)),
        out_shape=jax.ShapeDtypeStruct((_N, _N), jnp.float32),
        scratch_shapes=[pltpu.VMEM((_N, _D), jnp.float32),
                        pltpu.SemaphoreType.DMA],
    )(z_x)
